# baseline (speedup 1.0000x reference)
import jax
import jax.numpy as jnp
from jax.experimental import pallas as pl
from jax.experimental.pallas import tpu as pltpu

N = 4096
D_IN = 128
HIDDEN = 128
HEADS = 4
HEAD_DIM = HIDDEN // HEADS
HW = 128
ALPHA = 0.2
BR = 512
NEG = -1e9


def _ln(h, g, b, eps=1e-6):
    mu = jnp.mean(h, axis=-1, keepdims=True)
    var = jnp.mean((h - mu) ** 2, axis=-1, keepdims=True)
    return g * (h - mu) * jax.lax.rsqrt(var + eps) + b


NBLK = N // BR


def _block_kernel(x_ref, p_ref, w_ref, csrc_ref, cdst_ref,
                  ln1g_ref, ln1b_ref, ln2g_ref, ln2b_ref,
                  out_ref, h2_scr, fs_scr, v1_scr, v2_scr, fdm_scr, hm_scr,
                  r_scr, s_scr):
    i = pl.program_id(0)

    @pl.when(i == 0)
    def _prologue():
        xn = _ln(x_ref[...], ln1g_ref[0, :], ln1b_ref[0, :])
        fs_scr[...] = jnp.dot(xn, csrc_ref[...],
                              preferred_element_type=jnp.float32)
        fd = jax.lax.dot_general(
            cdst_ref[...], xn, (((1,), (1,)), ((), ())),
            preferred_element_type=jnp.float32)
        fdm = jnp.max(fd, axis=1, keepdims=True)
        fdm_scr[...] = fdm
        fd0 = fd - fdm
        v1_scr[...] = jnp.exp(fd0).astype(jnp.bfloat16)
        v2_scr[...] = jnp.exp(ALPHA * fd0).astype(jnp.bfloat16)
        for hd in range(HEADS):
            h_hd = jnp.dot(xn, w_ref[hd], preferred_element_type=jnp.float32)
            h2_scr[:, hd * HW:(hd + 1) * HW] = jnp.concatenate(
                [h_hd, jnp.ones((N, 1), jnp.float32),
                 jnp.zeros((N, HW - HEAD_DIM - 1), jnp.float32)],
                axis=1).astype(jnp.bfloat16)
            hm_scr[:, hd * HEAD_DIM:(hd + 1) * HEAD_DIM] = jnp.mean(
                h_hd, axis=0, keepdims=True)

    @pl.when(i < NBLK)
    def _matmuls():
        a01 = (p_ref[...] > 0.9).astype(jnp.bfloat16)
        rows = pl.ds(i * BR, BR)
        slot = i % 2
        for hd in range(HEADS):
            fsb = fs_scr[rows, hd:hd + 1]
            zm = fsb + fdm_scr[hd:hd + 1, 0:1]
            m = jnp.maximum(zm, ALPHA * zm)
            u1 = jnp.exp(zm - m).astype(jnp.bfloat16)
            u2 = jnp.exp(ALPHA * zm - m).astype(jnp.bfloat16)
            p = jnp.maximum(u1 * v1_scr[hd:hd + 1, :],
                            u2 * v2_scr[hd:hd + 1, :]) * a01
            r = jnp.dot(p, h2_scr[:, hd * HW:(hd + 1) * HW],
                        preferred_element_type=jnp.float32)
            r_scr[slot, :, hd * HEAD_DIM:(hd + 1) * HEAD_DIM] = r[:, :HEAD_DIM]
            s_scr[slot, :, hd:hd + 1] = r[:, HEAD_DIM:HEAD_DIM + 1]

    @pl.when(i > 0)
    def _epilogue():
        slot = (i - 1) % 2
        rows = pl.ds((i - 1) * BR, BR)
        outs = []
        for hd in range(HEADS):
            o = r_scr[slot, :, hd * HEAD_DIM:(hd + 1) * HEAD_DIM]
            s = s_scr[slot, :, hd:hd + 1]
            outs.append(jnp.where(s > 0, o / s,
                                  hm_scr[:, hd * HEAD_DIM:(hd + 1) * HEAD_DIM]))
        sub = jnp.concatenate(outs, axis=1)
        sub = jnp.where(sub > 0, sub, jnp.exp(sub) - 1.0)
        x2 = x_ref[rows, :] + sub
        out_ref[...] = _ln(x2, ln2g_ref[0, :], ln2b_ref[0, :])


@jax.jit
def kernel(x, mask, inputP, W, a_src, a_dst, ln1_g, ln1_b, ln2_g, ln2_b):
    del mask
    csrc = jnp.einsum('hdk,hk->dh', W, a_src)
    cdst = jnp.einsum('hdk,hk->hd', W, a_dst)
    grid = (N // BR + 1,)
    out = pl.pallas_call(
        _block_kernel,
        grid=grid,
        in_specs=[
            pl.BlockSpec((N, D_IN), lambda i: (0, 0)),
            pl.BlockSpec((BR, N),
                         lambda i: (jnp.minimum(i, N // BR - 1), 0)),
            pl.BlockSpec((HEADS, D_IN, HEAD_DIM), lambda i: (0, 0, 0)),
            pl.BlockSpec((D_IN, HEADS), lambda i: (0, 0)),
            pl.BlockSpec((HEADS, D_IN), lambda i: (0, 0)),
            pl.BlockSpec((1, D_IN), lambda i: (0, 0)),
            pl.BlockSpec((1, D_IN), lambda i: (0, 0)),
            pl.BlockSpec((1, HIDDEN), lambda i: (0, 0)),
            pl.BlockSpec((1, HIDDEN), lambda i: (0, 0)),
        ],
        compiler_params=pltpu.CompilerParams(
            vmem_limit_bytes=100 * 1024 * 1024),
        out_specs=pl.BlockSpec((BR, HIDDEN),
                               lambda i: (jnp.maximum(i - 1, 0), 0)),
        out_shape=jax.ShapeDtypeStruct((N, HIDDEN), jnp.float32),
        scratch_shapes=[
            pltpu.VMEM((N, HEADS * HW), jnp.bfloat16),
            pltpu.VMEM((N, HEADS), jnp.float32),
            pltpu.VMEM((HEADS, N), jnp.bfloat16),
            pltpu.VMEM((HEADS, N), jnp.bfloat16),
            pltpu.VMEM((HEADS, 1), jnp.float32),
            pltpu.VMEM((1, HIDDEN), jnp.float32),
            pltpu.VMEM((2, BR, HIDDEN), jnp.float32),
            pltpu.VMEM((2, BR, HEADS), jnp.float32),
        ],
    )(x, inputP, W, csrc, cdst,
      ln1_g.reshape(1, D_IN), ln1_b.reshape(1, D_IN),
      ln2_g.reshape(1, HIDDEN), ln2_b.reshape(1, HIDDEN))
    return out


# speedup vs baseline: 1.0357x; 1.0357x over previous
import jax
import jax.numpy as jnp
from jax.experimental import pallas as pl
from jax.experimental.pallas import tpu as pltpu

N = 4096
D_IN = 128
HIDDEN = 128
HEADS = 4
HEAD_DIM = HIDDEN // HEADS
HW = 128
ALPHA = 0.2
BR = 512
NEG = -1e9


def _ln(h, g, b, eps=1e-6):
    mu = jnp.mean(h, axis=-1, keepdims=True)
    var = jnp.mean((h - mu) ** 2, axis=-1, keepdims=True)
    return g * (h - mu) * jax.lax.rsqrt(var + eps) + b


def _block_kernel(x_ref, p_ref, w_ref, csrc_ref, cdst_ref,
                  ln1g_ref, ln1b_ref, ln2g_ref, ln2b_ref,
                  out_ref, h2_scr, fs_scr, v1_scr, v2_scr, fdm_scr, hm_scr):
    i = pl.program_id(0)

    @pl.when(i == 0)
    def _prologue():
        xn = _ln(x_ref[...], ln1g_ref[0, :], ln1b_ref[0, :])
        fs_scr[...] = jnp.dot(xn, csrc_ref[...],
                              preferred_element_type=jnp.float32)
        fd = jax.lax.dot_general(
            cdst_ref[...], xn, (((1,), (1,)), ((), ())),
            preferred_element_type=jnp.float32)
        fdm = jnp.max(fd, axis=1, keepdims=True)
        fdm_scr[...] = fdm
        fd0 = fd - fdm
        v1_scr[...] = jnp.exp(fd0).astype(jnp.bfloat16)
        v2_scr[...] = jnp.exp(ALPHA * fd0).astype(jnp.bfloat16)
        for hd in range(HEADS):
            h_hd = jnp.dot(xn, w_ref[hd], preferred_element_type=jnp.float32)
            h2_scr[:, hd * HW:(hd + 1) * HW] = jnp.concatenate(
                [h_hd, jnp.ones((N, 1), jnp.float32),
                 jnp.zeros((N, HW - HEAD_DIM - 1), jnp.float32)],
                axis=1).astype(jnp.bfloat16)
            hm_scr[:, hd * HEAD_DIM:(hd + 1) * HEAD_DIM] = jnp.mean(
                h_hd, axis=0, keepdims=True)

    a01 = (p_ref[...] > 0.9).astype(jnp.bfloat16)
    rows = pl.ds(i * BR, BR)
    outs = []
    for hd in range(HEADS):
        fsb = fs_scr[rows, hd:hd + 1]
        zm = fsb + fdm_scr[hd:hd + 1, 0:1]
        m = jnp.maximum(zm, ALPHA * zm)
        u1 = jnp.exp(zm - m).astype(jnp.bfloat16)
        u2 = jnp.exp(ALPHA * zm - m).astype(jnp.bfloat16)
        p = jnp.maximum(u1 * v1_scr[hd:hd + 1, :],
                        u2 * v2_scr[hd:hd + 1, :]) * a01
        r = jnp.dot(p, h2_scr[:, hd * HW:(hd + 1) * HW],
                    preferred_element_type=jnp.float32)
        o = r[:, :HEAD_DIM]
        s = r[:, HEAD_DIM:HEAD_DIM + 1]
        outs.append(jnp.where(s > 0, o / s,
                              hm_scr[:, hd * HEAD_DIM:(hd + 1) * HEAD_DIM]))
    sub = jnp.concatenate(outs, axis=1)
    sub = jnp.where(sub > 0, sub, jnp.exp(sub) - 1.0)
    x2 = x_ref[rows, :] + sub
    out_ref[...] = _ln(x2, ln2g_ref[0, :], ln2b_ref[0, :])


@jax.jit
def kernel(x, mask, inputP, W, a_src, a_dst, ln1_g, ln1_b, ln2_g, ln2_b):
    del mask
    csrc = jnp.einsum('hdk,hk->dh', W, a_src)
    cdst = jnp.einsum('hdk,hk->hd', W, a_dst)
    grid = (N // BR,)
    out = pl.pallas_call(
        _block_kernel,
        grid=grid,
        in_specs=[
            pl.BlockSpec((N, D_IN), lambda i: (0, 0)),
            pl.BlockSpec((BR, N), lambda i: (i, 0)),
            pl.BlockSpec((HEADS, D_IN, HEAD_DIM), lambda i: (0, 0, 0)),
            pl.BlockSpec((D_IN, HEADS), lambda i: (0, 0)),
            pl.BlockSpec((HEADS, D_IN), lambda i: (0, 0)),
            pl.BlockSpec((1, D_IN), lambda i: (0, 0)),
            pl.BlockSpec((1, D_IN), lambda i: (0, 0)),
            pl.BlockSpec((1, HIDDEN), lambda i: (0, 0)),
            pl.BlockSpec((1, HIDDEN), lambda i: (0, 0)),
        ],
        compiler_params=pltpu.CompilerParams(
            vmem_limit_bytes=100 * 1024 * 1024),
        out_specs=pl.BlockSpec((BR, HIDDEN), lambda i: (i, 0)),
        out_shape=jax.ShapeDtypeStruct((N, HIDDEN), jnp.float32),
        scratch_shapes=[
            pltpu.VMEM((N, HEADS * HW), jnp.bfloat16),
            pltpu.VMEM((N, HEADS), jnp.float32),
            pltpu.VMEM((HEADS, N), jnp.bfloat16),
            pltpu.VMEM((HEADS, N), jnp.bfloat16),
            pltpu.VMEM((HEADS, 1), jnp.float32),
            pltpu.VMEM((1, HIDDEN), jnp.float32),
        ],
    )(x, inputP, W, csrc, cdst,
      ln1_g.reshape(1, D_IN), ln1_b.reshape(1, D_IN),
      ln2_g.reshape(1, HIDDEN), ln2_b.reshape(1, HIDDEN))
    return out
